# baseline (speedup 1.0000x reference)
import functools

import jax
import jax.numpy as jnp
from jax import lax
from jax.experimental import pallas as pl
from jax.experimental.pallas import tpu as pltpu
from jax.experimental.pallas import tpu_sc as plsc

T, V, D, L = 16384, 100000, 128, 64
_NC, _NS = 2, 16
_NW = _NC * _NS
_BPW = T // _NW
_CH = _BPW // 128


def _sc_body(idx_hbm, gaz_hbm, table_hbm, out_hbm,
             idx_v, rows_v, gz0, gz1, semG, semZ0, semZ1, semO):
    wid = lax.axis_index("s") * _NC + lax.axis_index("c")
    base = wid * _BPW
    gzb = (gz0, gz1)
    semZ = (semZ0, semZ1)

    pltpu.sync_copy(idx_hbm.at[wid], idx_v)
    gathers = [
        pltpu.async_copy(
            table_hbm.at[idx_v.at[j]],
            rows_v.at[pl.ds(j * 128, 128)],
            semG,
        )
        for j in range(_CH)
    ]
    z = pltpu.async_copy(gaz_hbm.at[pl.ds(base, 128)], gz0, semZ0)
    zc = {0: z}
    for g in gathers:
        g.wait()

    outs = []
    for j in range(_CH):
        if j + 1 < _CH:
            b = (j + 1) % 2
            zc[j + 1] = pltpu.async_copy(
                gaz_hbm.at[pl.ds(base + (j + 1) * 128, 128)], gzb[b], semZ[b])
        zc[j].wait()
        buf = gzb[j % 2]

        def add_row(r, _, j=j, buf=buf):
            for c in range(8):
                sl = pl.ds(c * 16, 16)
                plsc.addupdate(rows_v.at[j * 128 + r, sl], buf[r, sl])
            return _

        lax.fori_loop(0, 128, add_row, None)
        outs.append(
            pltpu.async_copy(
                rows_v.at[pl.ds(j * 128, 128)],
                out_hbm.at[pl.ds(base + j * 128, 128)],
                semO,
            )
        )
    for o in outs:
        o.wait()


_sc_gather_add = pl.kernel(
    _sc_body,
    out_type=jax.ShapeDtypeStruct((T, D), jnp.float32),
    mesh=plsc.VectorSubcoreMesh(core_axis_name="c", subcore_axis_name="s"),
    scratch_types=[
        pltpu.VMEM((_CH, 128), jnp.int32),
        pltpu.VMEM((_BPW, D), jnp.float32),
        pltpu.VMEM((128, D), jnp.float32),
        pltpu.VMEM((128, D), jnp.float32),
        pltpu.SemaphoreType.DMA,
        pltpu.SemaphoreType.DMA,
        pltpu.SemaphoreType.DMA,
        pltpu.SemaphoreType.DMA,
    ],
)


def _mm_body(g_ref, wt_ref, b0_ref, b1_ref, out_ref):
    acc = jnp.dot(g_ref[...], wt_ref[...], preferred_element_type=jnp.float32)
    out_ref[...] = acc + b0_ref[...] + b1_ref[...]


def _tc_matmul(g, wt, b0, b1):
    bT = 4096
    return pl.pallas_call(
        _mm_body,
        out_shape=jax.ShapeDtypeStruct((T, D), jnp.float32),
        grid=(T // bT,),
        in_specs=[
            pl.BlockSpec((bT, 2 * L), lambda i: (i, 0)),
            pl.BlockSpec((2 * L, D), lambda i: (0, 0)),
            pl.BlockSpec((1, D), lambda i: (0, 0)),
            pl.BlockSpec((1, D), lambda i: (0, 0)),
        ],
        out_specs=pl.BlockSpec((bT, D), lambda i: (i, 0)),
    )(g, wt, b0, b1)


def kernel(sentence_data, batch_sizes, gazetteers_data, word_table, W0, b0, W1, b1):
    del batch_sizes
    idx = sentence_data.reshape(_NW, _CH, 128)
    wt = jnp.concatenate([W0.T, W1.T], axis=0)
    gaz = _tc_matmul(gazetteers_data, wt, b0[None, :], b1[None, :])
    return _sc_gather_add(idx, gaz, word_table)


# speedup vs baseline: 1.0351x; 1.0351x over previous
import functools

import jax
import jax.numpy as jnp
from jax import lax
from jax.experimental import pallas as pl
from jax.experimental.pallas import tpu as pltpu
from jax.experimental.pallas import tpu_sc as plsc

T, V, D, L = 16384, 100000, 128, 64
_NC, _NS = 2, 16
_NW = _NC * _NS
_BPW = T // _NW
_CH = _BPW // 128


def _sc_body(idx_hbm, table_hbm, out_hbm, idx_v, rows_v,
             sg0, sg1, sg2, sg3, semO):
    wid = lax.axis_index("s") * _NC + lax.axis_index("c")
    base = wid * _BPW
    sems = (sg0, sg1, sg2, sg3)

    pltpu.sync_copy(idx_hbm.at[wid], idx_v)
    gathers = [
        pltpu.async_copy(
            table_hbm.at[idx_v.at[j]],
            rows_v.at[pl.ds(j * 128, 128)],
            sems[j],
        )
        for j in range(_CH)
    ]
    outs = []
    for j in range(_CH):
        gathers[j].wait()
        outs.append(
            pltpu.async_copy(
                rows_v.at[pl.ds(j * 128, 128)],
                out_hbm.at[pl.ds(base + j * 128, 128)],
                semO,
            )
        )
    for o in outs:
        o.wait()


_sc_gather = pl.kernel(
    _sc_body,
    out_type=jax.ShapeDtypeStruct((T, D), jnp.float32),
    mesh=plsc.VectorSubcoreMesh(core_axis_name="c", subcore_axis_name="s"),
    scratch_types=[
        pltpu.VMEM((_CH, 128), jnp.int32),
        pltpu.VMEM((_BPW, D), jnp.float32),
        pltpu.SemaphoreType.DMA,
        pltpu.SemaphoreType.DMA,
        pltpu.SemaphoreType.DMA,
        pltpu.SemaphoreType.DMA,
        pltpu.SemaphoreType.DMA,
    ],
)


def _tc_body(g_ref, wemb_ref, wt_ref, b0_ref, b1_ref, out_ref):
    acc = jnp.dot(g_ref[...], wt_ref[...], preferred_element_type=jnp.float32)
    out_ref[...] = wemb_ref[...] + acc + b0_ref[...] + b1_ref[...]


def _tc_matmul_add(g, wemb, wt, b0, b1):
    bT = 4096
    return pl.pallas_call(
        _tc_body,
        out_shape=jax.ShapeDtypeStruct((T, D), jnp.float32),
        grid=(T // bT,),
        in_specs=[
            pl.BlockSpec((bT, 2 * L), lambda i: (i, 0)),
            pl.BlockSpec((bT, D), lambda i: (i, 0)),
            pl.BlockSpec((2 * L, D), lambda i: (0, 0)),
            pl.BlockSpec((1, D), lambda i: (0, 0)),
            pl.BlockSpec((1, D), lambda i: (0, 0)),
        ],
        out_specs=pl.BlockSpec((bT, D), lambda i: (i, 0)),
    )(g, wemb, wt, b0, b1)


def kernel(sentence_data, batch_sizes, gazetteers_data, word_table, W0, b0, W1, b1):
    del batch_sizes
    idx = sentence_data.reshape(_NW, _CH, 128)
    wemb = _sc_gather(idx, word_table)
    wt = jnp.concatenate([W0.T, W1.T], axis=0)
    return _tc_matmul_add(gazetteers_data, wemb, wt, b0[None, :], b1[None, :])
